# baseline (speedup 1.0000x reference)
import functools

import jax
import jax.numpy as jnp
from jax import lax
from jax.experimental import pallas as pl
from jax.experimental.pallas import tpu as pltpu
from jax.experimental.pallas import tpu_sc as plsc

VOCAB = 1000000
DIM = 64
B = 16384
NEG = 20
NU = NEG + 1
NUP = 32
L = 16

NC = 2
NS = 16
NW = NC * NS
BPW = B // NW

C = 64
NCHUNK = BPW // C
SLEN = 112
NSTREAM = (C * NU) // SLEN
assert NSTREAM * SLEN == C * NU


TBL = 512
TAIL0 = (VOCAB // TBL) * TBL


def _tc_tr_body(s_ref, o_ref):
    eye = jnp.eye(DIM, dtype=jnp.float32)
    o_ref[...] = lax.dot_general(
        s_ref[...], eye, (((0,), (0,)), ((), ())),
        preferred_element_type=jnp.float32)


def _tc_convert(xt):
    return pl.pallas_call(
        _tc_tr_body,
        grid=(pl.cdiv(VOCAB, TBL),),
        in_specs=[pl.BlockSpec((DIM, TBL), lambda i: (0, i))],
        out_specs=pl.BlockSpec((TBL, DIM), lambda i: (i, 0)),
        out_shape=jax.ShapeDtypeStruct((VOCAB, DIM), jnp.float32),
    )(xt)


def _hsum_vec(qs, iota):
    acc = jnp.zeros((L,), jnp.float32)
    for k, q in enumerate(qs):
        acc = jnp.where(iota == k, jnp.sum(q), acc)
    return acc


def _sc_scores():
    mesh = plsc.VectorSubcoreMesh(core_axis_name="c", subcore_axis_name="s")

    @functools.partial(
        pl.kernel,
        mesh=mesh,
        compiler_params=pltpu.CompilerParams(
            needs_layout_passes=False, use_tc_tiling_on_sc=False),
        out_type=jax.ShapeDtypeStruct((B, NUP), jnp.float32),
        scratch_types=[
            pltpu.VMEM((BPW,), jnp.int32),
            pltpu.VMEM((BPW * NU,), jnp.int32),
            pltpu.VMEM((C, DIM), jnp.float32),
            pltpu.VMEM((C * NU, DIM), jnp.float32),
            pltpu.VMEM((C, NUP), jnp.float32),
            pltpu.SemaphoreType.DMA,
        ],
    )
    def k(centers_hbm, idxu_hbm, v_hbm, u_hbm, out_hbm,
          idxc_v, idxu_v, vc_v, ur_v, sc_v, sem):
        wid = lax.axis_index("s") * NC + lax.axis_index("c")
        base = wid * BPW
        pltpu.sync_copy(centers_hbm.at[pl.ds(base, BPW)], idxc_v)
        pltpu.sync_copy(idxu_hbm.at[pl.ds(base * NU, BPW * NU)], idxu_v)

        iota = lax.iota(jnp.int32, L)

        def chunk_body(ci, carry):
            cb = ci * C
            cps = [pltpu.async_copy(v_hbm.at[idxc_v.at[pl.ds(cb, C)]], vc_v, sem)]
            for j in range(NSTREAM):
                cps.append(pltpu.async_copy(
                    u_hbm.at[idxu_v.at[pl.ds(cb * NU + j * SLEN, SLEN)]],
                    ur_v.at[pl.ds(j * SLEN, SLEN)], sem))
            for cp in cps:
                cp.wait()

            def elem(b, carry2):
                a0 = vc_v[b, pl.ds(0, L)]
                a1 = vc_v[b, pl.ds(L, L)]
                a2 = vc_v[b, pl.ds(2 * L, L)]
                a3 = vc_v[b, pl.ds(3 * L, L)]
                r0 = b * NU
                qs = []
                for kk in range(NU):
                    qs.append(a0 * ur_v[r0 + kk, pl.ds(0, L)]
                              + a1 * ur_v[r0 + kk, pl.ds(L, L)]
                              + a2 * ur_v[r0 + kk, pl.ds(2 * L, L)]
                              + a3 * ur_v[r0 + kk, pl.ds(3 * L, L)])
                sc_v[b, pl.ds(0, L)] = _hsum_vec(qs[:L], iota)
                sc_v[b, pl.ds(L, L)] = _hsum_vec(qs[L:], iota)
                return carry2

            lax.fori_loop(0, C, elem, 0)
            pltpu.sync_copy(sc_v, out_hbm.at[pl.ds(base + cb, C)])
            return carry

        lax.fori_loop(0, NCHUNK, chunk_body, 0)

    return k


_SC_SCORES = _sc_scores()

ROWS = (B * NUP) // 128


def _loss_body(s_ref, o_ref):
    s = s_ref[:]
    col = lax.broadcasted_iota(jnp.int32, (ROWS, 128), 1) % NUP
    is_pos = col == 0
    valid = col < NU
    t = jnp.where(is_pos, s, -s)
    term = jnp.where(valid, -jnp.log(jax.nn.sigmoid(t) + 1e-12), 0.0)
    o_ref[0, 0] = jnp.sum(term) * (1.0 / B)


def kernel(centers, pos, neg, V, U):
    centers = centers.astype(jnp.int32)
    idxu = jnp.concatenate(
        [pos.astype(jnp.int32)[:, None], neg.astype(jnp.int32)], axis=1
    ).reshape(-1)
    Vlin = _tc_convert(V.T)
    Ulin = _tc_convert(U.T)
    scores = _SC_SCORES(centers, idxu, Vlin, Ulin)
    s2 = scores.reshape(ROWS, 128)
    loss = pl.pallas_call(
        _loss_body,
        out_shape=jax.ShapeDtypeStruct((1, 1), jnp.float32),
        out_specs=pl.BlockSpec(memory_space=pltpu.SMEM),
    )(s2)
    return loss[0, 0]


# speedup vs baseline: 2.1969x; 2.1969x over previous
import functools

import jax
import jax.numpy as jnp
from jax import lax
from jax.experimental import pallas as pl
from jax.experimental.pallas import tpu as pltpu
from jax.experimental.pallas import tpu_sc as plsc

VOCAB = 1000000
DIM = 64
B = 16384
NEG = 20
NU = NEG + 1
NUP = 32
L = 16

NC = 2
NS = 16
NW = NC * NS
BPW = B // NW

C = 64
NCHUNK = BPW // C
SLEN = 112
NSTREAM = (C * NU) // SLEN
assert NSTREAM * SLEN == C * NU


TBL = 4096
TAIL0 = (VOCAB // TBL) * TBL


def _tc_tr_body(s_ref, o_ref):
    eye = jnp.eye(DIM, dtype=jnp.float32)
    o_ref[...] = lax.dot_general(
        s_ref[...], eye, (((0,), (0,)), ((), ())),
        preferred_element_type=jnp.float32)


def _tc_convert(xt):
    return pl.pallas_call(
        _tc_tr_body,
        grid=(pl.cdiv(VOCAB, TBL),),
        in_specs=[pl.BlockSpec((DIM, TBL), lambda i: (0, i))],
        out_specs=pl.BlockSpec((TBL, DIM), lambda i: (i, 0)),
        out_shape=jax.ShapeDtypeStruct((VOCAB, DIM), jnp.float32),
    )(xt)


def _hsum_vec(qs, iota):
    acc = jnp.zeros((L,), jnp.float32)
    for k, q in enumerate(qs):
        acc = jnp.where(iota == k, jnp.sum(q), acc)
    return acc


def _sc_scores():
    mesh = plsc.VectorSubcoreMesh(core_axis_name="c", subcore_axis_name="s")

    @functools.partial(
        pl.kernel,
        mesh=mesh,
        compiler_params=pltpu.CompilerParams(
            needs_layout_passes=False, use_tc_tiling_on_sc=False),
        out_type=jax.ShapeDtypeStruct((B, NUP), jnp.float32),
        scratch_types=[
            pltpu.VMEM((BPW,), jnp.int32),
            pltpu.VMEM((BPW * NU,), jnp.int32),
            pltpu.VMEM((C, DIM), jnp.float32),
            pltpu.VMEM((C * NU, DIM), jnp.float32),
            pltpu.VMEM((C, NUP), jnp.float32),
            pltpu.SemaphoreType.DMA,
        ],
    )
    def k(centers_hbm, idxu_hbm, v_hbm, u_hbm, out_hbm,
          idxc_v, idxu_v, vc_v, ur_v, sc_v, sem):
        wid = lax.axis_index("s") * NC + lax.axis_index("c")
        base = wid * BPW
        pltpu.sync_copy(centers_hbm.at[pl.ds(base, BPW)], idxc_v)
        pltpu.sync_copy(idxu_hbm.at[pl.ds(base * NU, BPW * NU)], idxu_v)

        iota = lax.iota(jnp.int32, L)

        def chunk_body(ci, carry):
            cb = ci * C
            cps = [pltpu.async_copy(v_hbm.at[idxc_v.at[pl.ds(cb, C)]], vc_v, sem)]
            for j in range(NSTREAM):
                cps.append(pltpu.async_copy(
                    u_hbm.at[idxu_v.at[pl.ds(cb * NU + j * SLEN, SLEN)]],
                    ur_v.at[pl.ds(j * SLEN, SLEN)], sem))
            for cp in cps:
                cp.wait()

            def elem(b, carry2):
                a0 = vc_v[b, pl.ds(0, L)]
                a1 = vc_v[b, pl.ds(L, L)]
                a2 = vc_v[b, pl.ds(2 * L, L)]
                a3 = vc_v[b, pl.ds(3 * L, L)]
                r0 = b * NU
                qs = []
                for kk in range(NU):
                    qs.append(a0 * ur_v[r0 + kk, pl.ds(0, L)]
                              + a1 * ur_v[r0 + kk, pl.ds(L, L)]
                              + a2 * ur_v[r0 + kk, pl.ds(2 * L, L)]
                              + a3 * ur_v[r0 + kk, pl.ds(3 * L, L)])
                sc_v[b, pl.ds(0, L)] = _hsum_vec(qs[:L], iota)
                sc_v[b, pl.ds(L, L)] = _hsum_vec(qs[L:], iota)
                return carry2

            lax.fori_loop(0, C, elem, 0)
            pltpu.sync_copy(sc_v, out_hbm.at[pl.ds(base + cb, C)])
            return carry

        lax.fori_loop(0, NCHUNK, chunk_body, 0)

    return k


_SC_SCORES = _sc_scores()

ROWS = (B * NUP) // 128


def _loss_body(s_ref, o_ref):
    s = s_ref[:]
    col = lax.broadcasted_iota(jnp.int32, (ROWS, 128), 1) % NUP
    is_pos = col == 0
    valid = col < NU
    t = jnp.where(is_pos, s, -s)
    term = jnp.where(valid, -jnp.log(jax.nn.sigmoid(t) + 1e-12), 0.0)
    o_ref[0, 0] = jnp.sum(term) * (1.0 / B)


def kernel(centers, pos, neg, V, U):
    centers = centers.astype(jnp.int32)
    idxu = jnp.concatenate(
        [pos.astype(jnp.int32)[:, None], neg.astype(jnp.int32)], axis=1
    ).reshape(-1)
    Vlin = _tc_convert(V.T)
    Ulin = _tc_convert(U.T)
    scores = _SC_SCORES(centers, idxu, Vlin, Ulin)
    s2 = scores.reshape(ROWS, 128)
    loss = pl.pallas_call(
        _loss_body,
        out_shape=jax.ShapeDtypeStruct((1, 1), jnp.float32),
        out_specs=pl.BlockSpec(memory_space=pltpu.SMEM),
    )(s2)
    return loss[0, 0]


# speedup vs baseline: 2.5262x; 1.1499x over previous
import functools

import jax
import jax.numpy as jnp
from jax import lax
from jax.experimental import pallas as pl
from jax.experimental.pallas import tpu as pltpu
from jax.experimental.pallas import tpu_sc as plsc

VOCAB = 1000000
DIM = 64
B = 16384
NEG = 20
NU = NEG + 1
NUP = 32
L = 16

NC = 2
NS = 16
NW = NC * NS
BPW = B // NW

C = 64
NCHUNK = BPW // C
SLEN = 112
NSTREAM = (C * NU) // SLEN
assert NSTREAM * SLEN == C * NU


TBL = 16384
TAIL0 = (VOCAB // TBL) * TBL


def _tc_tr_body(s_ref, o_ref):
    eye = jnp.eye(DIM, dtype=jnp.float32)
    o_ref[...] = lax.dot_general(
        s_ref[...], eye, (((0,), (0,)), ((), ())),
        preferred_element_type=jnp.float32)


def _tc_convert(xt):
    return pl.pallas_call(
        _tc_tr_body,
        grid=(pl.cdiv(VOCAB, TBL),),
        in_specs=[pl.BlockSpec((DIM, TBL), lambda i: (0, i))],
        out_specs=pl.BlockSpec((TBL, DIM), lambda i: (i, 0)),
        out_shape=jax.ShapeDtypeStruct((VOCAB, DIM), jnp.float32),
    )(xt)


def _hsum_vec(qs, iota):
    acc = jnp.zeros((L,), jnp.float32)
    for k, q in enumerate(qs):
        acc = jnp.where(iota == k, jnp.sum(q), acc)
    return acc


def _sc_scores():
    mesh = plsc.VectorSubcoreMesh(core_axis_name="c", subcore_axis_name="s")

    @functools.partial(
        pl.kernel,
        mesh=mesh,
        compiler_params=pltpu.CompilerParams(
            needs_layout_passes=False, use_tc_tiling_on_sc=False),
        out_type=jax.ShapeDtypeStruct((B, NUP), jnp.float32),
        scratch_types=[
            pltpu.VMEM((BPW,), jnp.int32),
            pltpu.VMEM((BPW * NU,), jnp.int32),
            pltpu.VMEM((C, DIM), jnp.float32),
            pltpu.VMEM((C * NU, DIM), jnp.float32),
            pltpu.VMEM((C, NUP), jnp.float32),
            pltpu.SemaphoreType.DMA,
        ],
    )
    def k(centers_hbm, idxu_hbm, v_hbm, u_hbm, out_hbm,
          idxc_v, idxu_v, vc_v, ur_v, sc_v, sem):
        wid = lax.axis_index("s") * NC + lax.axis_index("c")
        base = wid * BPW
        pltpu.sync_copy(centers_hbm.at[pl.ds(base, BPW)], idxc_v)
        pltpu.sync_copy(idxu_hbm.at[pl.ds(base * NU, BPW * NU)], idxu_v)

        iota = lax.iota(jnp.int32, L)

        def chunk_body(ci, carry):
            cb = ci * C
            cps = [pltpu.async_copy(v_hbm.at[idxc_v.at[pl.ds(cb, C)]], vc_v, sem)]
            for j in range(NSTREAM):
                cps.append(pltpu.async_copy(
                    u_hbm.at[idxu_v.at[pl.ds(cb * NU + j * SLEN, SLEN)]],
                    ur_v.at[pl.ds(j * SLEN, SLEN)], sem))
            for cp in cps:
                cp.wait()

            def elem(b, carry2):
                a0 = vc_v[b, pl.ds(0, L)]
                a1 = vc_v[b, pl.ds(L, L)]
                a2 = vc_v[b, pl.ds(2 * L, L)]
                a3 = vc_v[b, pl.ds(3 * L, L)]
                r0 = b * NU
                qs = []
                for kk in range(NU):
                    qs.append(a0 * ur_v[r0 + kk, pl.ds(0, L)]
                              + a1 * ur_v[r0 + kk, pl.ds(L, L)]
                              + a2 * ur_v[r0 + kk, pl.ds(2 * L, L)]
                              + a3 * ur_v[r0 + kk, pl.ds(3 * L, L)])
                sc_v[b, pl.ds(0, L)] = _hsum_vec(qs[:L], iota)
                sc_v[b, pl.ds(L, L)] = _hsum_vec(qs[L:], iota)
                return carry2

            lax.fori_loop(0, C, elem, 0)
            pltpu.sync_copy(sc_v, out_hbm.at[pl.ds(base + cb, C)])
            return carry

        lax.fori_loop(0, NCHUNK, chunk_body, 0)

    return k


_SC_SCORES = _sc_scores()

ROWS = (B * NUP) // 128


def _loss_body(s_ref, o_ref):
    s = s_ref[:]
    col = lax.broadcasted_iota(jnp.int32, (ROWS, 128), 1) % NUP
    is_pos = col == 0
    valid = col < NU
    t = jnp.where(is_pos, s, -s)
    term = jnp.where(valid, -jnp.log(jax.nn.sigmoid(t) + 1e-12), 0.0)
    o_ref[0, 0] = jnp.sum(term) * (1.0 / B)


def kernel(centers, pos, neg, V, U):
    centers = centers.astype(jnp.int32)
    idxu = jnp.concatenate(
        [pos.astype(jnp.int32)[:, None], neg.astype(jnp.int32)], axis=1
    ).reshape(-1)
    Vlin = _tc_convert(V.T)
    Ulin = _tc_convert(U.T)
    scores = _SC_SCORES(centers, idxu, Vlin, Ulin)
    s2 = scores.reshape(ROWS, 128)
    loss = pl.pallas_call(
        _loss_body,
        out_shape=jax.ShapeDtypeStruct((1, 1), jnp.float32),
        out_specs=pl.BlockSpec(memory_space=pltpu.SMEM),
    )(s2)
    return loss[0, 0]


# speedup vs baseline: 2.6543x; 1.0507x over previous
import functools

import jax
import jax.numpy as jnp
from jax import lax
from jax.experimental import pallas as pl
from jax.experimental.pallas import tpu as pltpu
from jax.experimental.pallas import tpu_sc as plsc

VOCAB = 1000000
DIM = 64
B = 16384
NEG = 20
NU = NEG + 1
NUP = 32
L = 16

NC = 2
NS = 16
NW = NC * NS
BPW = B // NW

C = 64
NCHUNK = BPW // C
SLEN = 112
NSTREAM = (C * NU) // SLEN
assert NSTREAM * SLEN == C * NU


TBL = 16384
TAIL0 = (VOCAB // TBL) * TBL


def _tc_tr_body(s_ref, o_ref):
    eye = jnp.eye(DIM, dtype=jnp.float32)
    o_ref[...] = lax.dot_general(
        s_ref[...], eye, (((0,), (0,)), ((), ())),
        preferred_element_type=jnp.float32)


def _tc_convert(xt):
    return pl.pallas_call(
        _tc_tr_body,
        grid=(pl.cdiv(VOCAB, TBL),),
        in_specs=[pl.BlockSpec((DIM, TBL), lambda i: (0, i))],
        out_specs=pl.BlockSpec((TBL, DIM), lambda i: (i, 0)),
        out_shape=jax.ShapeDtypeStruct((VOCAB, DIM), jnp.float32),
    )(xt)


def _hsum_vec(qs, iota):
    acc = jnp.zeros((L,), jnp.float32)
    for k, q in enumerate(qs):
        acc = jnp.where(iota == k, jnp.sum(q), acc)
    return acc


def _sc_scores():
    mesh = plsc.VectorSubcoreMesh(core_axis_name="c", subcore_axis_name="s")

    @functools.partial(
        pl.kernel,
        mesh=mesh,
        compiler_params=pltpu.CompilerParams(
            needs_layout_passes=False, use_tc_tiling_on_sc=False),
        out_type=jax.ShapeDtypeStruct((B, NUP), jnp.float32),
        scratch_types=[
            pltpu.VMEM((BPW,), jnp.int32),
            pltpu.VMEM((BPW * NU,), jnp.int32),
            pltpu.VMEM((C, DIM), jnp.float32),
            pltpu.VMEM((C * NU, DIM), jnp.float32),
            pltpu.VMEM((C, NUP), jnp.float32),
            pltpu.SemaphoreType.DMA,
        ],
    )
    def k(centers_hbm, idxu_hbm, v_hbm, u_hbm, out_hbm,
          idxc_v, idxu_v, vc_v, ur_v, sc_v, sem):
        wid = lax.axis_index("s") * NC + lax.axis_index("c")
        base = wid * BPW
        pltpu.sync_copy(centers_hbm.at[pl.ds(base, BPW)], idxc_v)
        pltpu.sync_copy(idxu_hbm.at[pl.ds(base * NU, BPW * NU)], idxu_v)

        iota = lax.iota(jnp.int32, L)

        def chunk_body(ci, carry):
            cb = ci * C
            cps = [pltpu.async_copy(v_hbm.at[idxc_v.at[pl.ds(cb, C)]], vc_v, sem)]
            for j in range(NSTREAM):
                cps.append(pltpu.async_copy(
                    u_hbm.at[idxu_v.at[pl.ds(cb * NU + j * SLEN, SLEN)]],
                    ur_v.at[pl.ds(j * SLEN, SLEN)], sem))
            for cp in cps:
                cp.wait()

            def elem(b, carry2):
                a0 = vc_v[b, pl.ds(0, L)]
                a1 = vc_v[b, pl.ds(L, L)]
                a2 = vc_v[b, pl.ds(2 * L, L)]
                a3 = vc_v[b, pl.ds(3 * L, L)]
                r0 = b * NU
                qs = []
                for kk in range(NU):
                    qs.append(a0 * ur_v[r0 + kk, pl.ds(0, L)]
                              + a1 * ur_v[r0 + kk, pl.ds(L, L)]
                              + a2 * ur_v[r0 + kk, pl.ds(2 * L, L)]
                              + a3 * ur_v[r0 + kk, pl.ds(3 * L, L)])
                sc_v[b, pl.ds(0, L)] = _hsum_vec(qs[:L], iota)
                sc_v[b, pl.ds(L, L)] = _hsum_vec(qs[L:], iota)
                return carry2

            lax.fori_loop(0, C, elem, 0)
            pltpu.sync_copy(sc_v, out_hbm.at[pl.ds(base + cb, C)])
            return carry

        lax.fori_loop(0, NCHUNK, chunk_body, 0)

    return k


_SC_SCORES = _sc_scores()

ROWS = (B * NUP) // 128


def _loss_body(s_ref, o_ref):
    s = s_ref[:]
    col = lax.broadcasted_iota(jnp.int32, (ROWS, 128), 1) % NUP
    is_pos = col == 0
    valid = col < NU
    t = jnp.where(is_pos, s, -s)
    term = jnp.where(valid, -jnp.log(jax.nn.sigmoid(t) + 1e-12), 0.0)
    o_ref[0, 0] = jnp.sum(term) * (1.0 / B)


def kernel(centers, pos, neg, V, U):
    centers = centers.astype(jnp.int32)
    idxu = jnp.concatenate(
        [pos.astype(jnp.int32)[:, None], neg.astype(jnp.int32)], axis=1
    ).reshape(-1)
    Vlin = V
    Ulin = _tc_convert(U.T)
    scores = _SC_SCORES(centers, idxu, Vlin, Ulin)
    s2 = scores.reshape(ROWS, 128)
    loss = pl.pallas_call(
        _loss_body,
        out_shape=jax.ShapeDtypeStruct((1, 1), jnp.float32),
        out_specs=pl.BlockSpec(memory_space=pltpu.SMEM),
    )(s2)
    return loss[0, 0]


# speedup vs baseline: 2.9166x; 1.0989x over previous
import functools

import jax
import jax.numpy as jnp
from jax import lax
from jax.experimental import pallas as pl
from jax.experimental.pallas import tpu as pltpu
from jax.experimental.pallas import tpu_sc as plsc

VOCAB = 1000000
DIM = 64
B = 16384
NEG = 20
NU = NEG + 1
NUP = 32
L = 16

NC = 2
NS = 16
NW = NC * NS
BPW = B // NW

C = 64
NCHUNK = BPW // C
SLEN = 112
NSTREAM = (C * NU) // SLEN
assert NSTREAM * SLEN == C * NU


def _hsum_vec(qs, iota):
    acc = jnp.zeros((L,), jnp.float32)
    for k, q in enumerate(qs):
        acc = jnp.where(iota == k, jnp.sum(q), acc)
    return acc


def _sc_scores():
    mesh = plsc.VectorSubcoreMesh(core_axis_name="c", subcore_axis_name="s")

    @functools.partial(
        pl.kernel,
        mesh=mesh,
        compiler_params=pltpu.CompilerParams(
            needs_layout_passes=False, use_tc_tiling_on_sc=False),
        out_type=jax.ShapeDtypeStruct((B, NUP), jnp.float32),
        scratch_types=[
            pltpu.VMEM((BPW,), jnp.int32),
            pltpu.VMEM((BPW * NU,), jnp.int32),
            pltpu.VMEM((C, DIM), jnp.float32),
            pltpu.VMEM((C * NU, DIM), jnp.float32),
            pltpu.VMEM((C, NUP), jnp.float32),
            pltpu.SemaphoreType.DMA,
        ],
    )
    def k(centers_hbm, idxu_hbm, v_hbm, u_hbm, out_hbm,
          idxc_v, idxu_v, vc_v, ur_v, sc_v, sem):
        wid = lax.axis_index("s") * NC + lax.axis_index("c")
        base = wid * BPW
        pltpu.sync_copy(centers_hbm.at[pl.ds(base, BPW)], idxc_v)
        pltpu.sync_copy(idxu_hbm.at[pl.ds(base * NU, BPW * NU)], idxu_v)

        iota = lax.iota(jnp.int32, L)

        def chunk_body(ci, carry):
            cb = ci * C
            cps = [pltpu.async_copy(v_hbm.at[idxc_v.at[pl.ds(cb, C)]], vc_v, sem)]
            for j in range(NSTREAM):
                cps.append(pltpu.async_copy(
                    u_hbm.at[idxu_v.at[pl.ds(cb * NU + j * SLEN, SLEN)]],
                    ur_v.at[pl.ds(j * SLEN, SLEN)], sem))
            for cp in cps:
                cp.wait()

            def elem(b, carry2):
                a0 = vc_v[b, pl.ds(0, L)]
                a1 = vc_v[b, pl.ds(L, L)]
                a2 = vc_v[b, pl.ds(2 * L, L)]
                a3 = vc_v[b, pl.ds(3 * L, L)]
                r0 = b * NU
                qs = []
                for kk in range(NU):
                    qs.append(a0 * ur_v[r0 + kk, pl.ds(0, L)]
                              + a1 * ur_v[r0 + kk, pl.ds(L, L)]
                              + a2 * ur_v[r0 + kk, pl.ds(2 * L, L)]
                              + a3 * ur_v[r0 + kk, pl.ds(3 * L, L)])
                sc_v[b, pl.ds(0, L)] = _hsum_vec(qs[:L], iota)
                sc_v[b, pl.ds(L, L)] = _hsum_vec(qs[L:], iota)
                return carry2

            lax.fori_loop(0, C, elem, 0)
            pltpu.sync_copy(sc_v, out_hbm.at[pl.ds(base + cb, C)])
            return carry

        lax.fori_loop(0, NCHUNK, chunk_body, 0)

    return k


_SC_SCORES = _sc_scores()

ROWS = (B * NUP) // 128


def _loss_body(s_ref, o_ref):
    s = s_ref[:]
    col = lax.broadcasted_iota(jnp.int32, (ROWS, 128), 1) % NUP
    is_pos = col == 0
    valid = col < NU
    t = jnp.where(is_pos, s, -s)
    term = jnp.where(valid, -jnp.log(jax.nn.sigmoid(t) + 1e-12), 0.0)
    o_ref[0, 0] = jnp.sum(term) * (1.0 / B)


def kernel(centers, pos, neg, V, U):
    centers = centers.astype(jnp.int32)
    idxu = jnp.concatenate(
        [pos.astype(jnp.int32)[:, None], neg.astype(jnp.int32)], axis=1
    ).reshape(-1)
    scores = _SC_SCORES(centers, idxu, V, U)
    s2 = scores.reshape(ROWS, 128)
    loss = pl.pallas_call(
        _loss_body,
        out_shape=jax.ShapeDtypeStruct((1, 1), jnp.float32),
        out_specs=pl.BlockSpec(memory_space=pltpu.SMEM),
    )(s2)
    return loss[0, 0]
